# Initial kernel scaffold; baseline (speedup 1.0000x reference)
import jax
import jax.numpy as jnp
from jax.experimental import pallas as pl


def kernel(x, edge_index, W_in_self, W_in_nbr, Wb1_self, Wb1_nbr, Wb2_self, Wb2_nbr, bn_gamma, bn_beta, W_lin, b_lin, W_lin1, b_lin1, W_c1, b_c1, W_c2, b_c2):
    raise NotImplementedError("write your pallas kernel here")


# speedup vs baseline: 8.3080x; 8.3080x over previous
import functools

import jax
import jax.numpy as jnp
from jax import lax
from jax.experimental import pallas as pl
from jax.experimental.pallas import tpu as pltpu
from jax.experimental.pallas import tpu_sc as plsc

_N = 100000
_E = 1600000
_M = 32
_NC = 2
_NS = 16
_LANES = 128
_GPC = 8
_CHUNK = _LANES * _GPC
_EPAD = 1605632
_G = _EPAD // _LANES
_R = 1000
_NB = _N // _R
_NE = _N + _R
_NR = 100096
_RPT = _NR // _NS
_ZC = 368


def _make_segsum(split_edges: bool, table_rows: int):
  eps = _EPAD // (_NC * _NS) if split_edges else _EPAD // _NS
  nchunks = eps // _CHUNK
  gps = eps // _LANES
  mesh = plsc.VectorSubcoreMesh(core_axis_name="c", subcore_axis_name="s",
                                num_cores=_NC, num_subcores=_NS)

  def body(table, src_g, dst_g, out, src_v, dst_v, rows_v, acc, sem):
    c = lax.axis_index("c")
    s = lax.axis_index("s")

    def zrow(i, _):
      rows_v[i, :] = jnp.zeros((16,), jnp.float32)
      return 0
    lax.fori_loop(0, _ZC, zrow, 0)
    row0 = s * _RPT

    def zacc(j, _):
      pltpu.sync_copy(rows_v.at[pl.ds(0, _ZC)],
                      acc.at[pl.ds(row0 + j * _ZC, _ZC)])
      return 0
    lax.fori_loop(0, _RPT // _ZC, zacc, 0)
    plsc.subcore_barrier()

    gbase = (c * _NS + s) * gps if split_edges else s * gps

    def chunk(i, _):
      g0 = gbase + i * _GPC
      if split_edges:
        pltpu.sync_copy(src_g.at[pl.ds(g0, _GPC)], src_v)
      else:
        pltpu.sync_copy(src_g.at[c, pl.ds(g0, _GPC)], src_v)
      pltpu.sync_copy(dst_g.at[pl.ds(g0, _GPC)], dst_v)
      cps = [pltpu.async_copy(table.at[src_v.at[j]],
                              rows_v.at[pl.ds(j * _LANES, _LANES)], sem)
             for j in range(_GPC)]
      for j in range(_GPC):
        cps[j].wait()
      for j in range(_GPC):
        pltpu.sync_copy(rows_v.at[pl.ds(j * _LANES, _LANES)],
                        acc.at[dst_v.at[j]], add=True)
      return 0
    lax.fori_loop(0, nchunks, chunk, 0)
    plsc.subcore_barrier()
    pltpu.sync_copy(acc.at[pl.ds(row0, _RPT)], out.at[c, pl.ds(row0, _RPT)])

  idx_shape = (_G, _LANES) if split_edges else (_NC, _G, _LANES)
  del idx_shape
  return pl.kernel(
      body,
      out_type=jax.ShapeDtypeStruct((_NC, _NR, 16), jnp.float32),
      mesh=mesh,
      scratch_types=[
          pltpu.VMEM((_GPC, _LANES), jnp.int32),
          pltpu.VMEM((_GPC, _LANES), jnp.int32),
          pltpu.VMEM((_CHUNK, 16), jnp.float32),
          pltpu.VMEM_SHARED((_NR, 16), jnp.float32),
          pltpu.SemaphoreType.DMA,
      ],
      compiler_params=pltpu.CompilerParams(use_tc_tiling_on_sc=False),
  )


def _h0_body(x_ref, p_ref, ws_ref, wn_ref, h_ref):
  i = pl.program_id(0)

  @pl.when(i < _NB)
  def _():
    pb = p_ref[0] + p_ref[1]
    h_ref[...] = (
        jnp.dot(x_ref[...], ws_ref[...], preferred_element_type=jnp.float32,
                precision=lax.Precision.HIGHEST)
        + jnp.dot(pb, wn_ref[...], preferred_element_type=jnp.float32,
                precision=lax.Precision.HIGHEST))

  @pl.when(i >= _NB)
  def _():
    h_ref[...] = jnp.zeros_like(h_ref)


_h0_call = pl.pallas_call(
    _h0_body,
    grid=(_NE // _R,),
    in_specs=[
        pl.BlockSpec((_R, 3), lambda i: (jnp.minimum(i, _NB - 1), 0)),
        pl.BlockSpec((_NC, _R, 16), lambda i: (0, jnp.minimum(i, _NB - 1), 0)),
        pl.BlockSpec((3, _M), lambda i: (0, 0)),
        pl.BlockSpec((16, _M), lambda i: (0, 0)),
    ],
    out_specs=pl.BlockSpec((_R, _M), lambda i: (i, 0)),
    out_shape=jax.ShapeDtypeStruct((_NE, _M), jnp.float32),
)


def _make_blk(with_stats: bool):
  def body(h_ref, m_ref, ws_ref, wn_ref, o_ref, *rest):
    i = pl.program_id(0)
    if with_stats:
      st_ref = rest[0]

      @pl.when(i == 0)
      def _():
        st_ref[...] = jnp.zeros_like(st_ref)

    @pl.when(i < _NB)
    def _():
      hb = h_ref[...]
      m = jnp.concatenate([m_ref[0], m_ref[1]], axis=1)
      o_ref[...] = hb + jnp.maximum(
          jnp.dot(m, wn_ref[...], preferred_element_type=jnp.float32,
                precision=lax.Precision.HIGHEST)
          + jnp.dot(hb, ws_ref[...], preferred_element_type=jnp.float32,
                precision=lax.Precision.HIGHEST), 0.0)

    @pl.when(i >= _NB)
    def _():
      o_ref[...] = jnp.zeros_like(o_ref)

    if with_stats:
      ob = o_ref[...]
      ssum = jnp.sum(ob, axis=0)
      ssq = jnp.sum(ob * ob, axis=0)
      rest[0][...] += jnp.pad(jnp.stack([ssum, ssq]), ((0, 6), (0, 0)))

  in_specs = [
      pl.BlockSpec((_R, _M), lambda i: (i, 0)),
      pl.BlockSpec((_NC, _R, 16), lambda i: (0, jnp.minimum(i, _NB - 1), 0)),
      pl.BlockSpec((_M, _M), lambda i: (0, 0)),
      pl.BlockSpec((_M, _M), lambda i: (0, 0)),
  ]
  if with_stats:
    return pl.pallas_call(
        body,
        grid=(_NE // _R,),
        in_specs=in_specs,
        out_specs=[
            pl.BlockSpec((_R, _M), lambda i: (i, 0)),
            pl.BlockSpec((8, _M), lambda i: (0, 0)),
        ],
        out_shape=[
            jax.ShapeDtypeStruct((_NE, _M), jnp.float32),
            jax.ShapeDtypeStruct((8, _M), jnp.float32),
        ],
    )
  return pl.pallas_call(
      body,
      grid=(_NE // _R,),
      in_specs=in_specs,
      out_specs=pl.BlockSpec((_R, _M), lambda i: (i, 0)),
      out_shape=jax.ShapeDtypeStruct((_NE, _M), jnp.float32),
  )


def _head_body(h_ref, st_ref, g_ref, b_ref, wl_ref, bl_ref, w1_ref, b1_ref,
               wc1_ref, bc1_ref, wc2_ref, bc2_ref, y_ref, fv_ref, off_ref):
  st = st_ref[...]
  nf = jnp.float32(_N)
  mean = st[0:1, :] / nf
  var = st[1:2, :] / nf - mean * mean
  inv = lax.rsqrt(var + 1e-5)
  hb = (h_ref[...] - mean) * inv * g_ref[...] + b_ref[...]
  hb = jnp.maximum(hb, 0.0)
  y_ref[...] = jnp.dot(hb, wl_ref[...],
                       preferred_element_type=jnp.float32,
                precision=lax.Precision.HIGHEST) + bl_ref[...]
  fv = jnp.dot(hb, w1_ref[...],
               preferred_element_type=jnp.float32,
                precision=lax.Precision.HIGHEST) + b1_ref[...]
  fv_ref[...] = fv
  t = jnp.maximum(
      jnp.dot(fv, wc1_ref[...], preferred_element_type=jnp.float32,
                precision=lax.Precision.HIGHEST)
      + bc1_ref[...], 0.0)
  off_ref[...] = jnp.dot(t, wc2_ref[...],
                         preferred_element_type=jnp.float32,
                precision=lax.Precision.HIGHEST) + bc2_ref[...]


_head_call = pl.pallas_call(
    _head_body,
    grid=(_NB,),
    in_specs=[
        pl.BlockSpec((_R, _M), lambda i: (i, 0)),
        pl.BlockSpec((8, _M), lambda i: (0, 0)),
        pl.BlockSpec((1, _M), lambda i: (0, 0)),
        pl.BlockSpec((1, _M), lambda i: (0, 0)),
        pl.BlockSpec((_M, 20), lambda i: (0, 0)),
        pl.BlockSpec((1, 20), lambda i: (0, 0)),
        pl.BlockSpec((_M, _M), lambda i: (0, 0)),
        pl.BlockSpec((1, _M), lambda i: (0, 0)),
        pl.BlockSpec((_M, _M), lambda i: (0, 0)),
        pl.BlockSpec((1, _M), lambda i: (0, 0)),
        pl.BlockSpec((_M, 3), lambda i: (0, 0)),
        pl.BlockSpec((1, 3), lambda i: (0, 0)),
    ],
    out_specs=[
        pl.BlockSpec((_R, 20), lambda i: (i, 0)),
        pl.BlockSpec((_R, _M), lambda i: (i, 0)),
        pl.BlockSpec((_R, 3), lambda i: (i, 0)),
    ],
    out_shape=[
        jax.ShapeDtypeStruct((_N, 20), jnp.float32),
        jax.ShapeDtypeStruct((_N, _M), jnp.float32),
        jax.ShapeDtypeStruct((_N, 3), jnp.float32),
    ],
)

_make_segsum = functools.lru_cache(maxsize=None)(_make_segsum)


def _seg_split(table, src_g, dst_g):
  return _make_segsum(True, _N + 8)(table, src_g, dst_g)


def _seg_feat(table, src_g, dst_g):
  return _make_segsum(False, 2 * _NE)(table, src_g, dst_g)


_blk_call = _make_blk(with_stats=False)
_blk_stats_call = _make_blk(with_stats=True)


def kernel(x, edge_index, W_in_self, W_in_nbr, Wb1_self, Wb1_nbr, Wb2_self,
           Wb2_nbr, bn_gamma, bn_beta, W_lin, b_lin, W_lin1, b_lin1, W_c1,
           b_c1, W_c2, b_c2):
  src = edge_index[0]
  dst = edge_index[1]
  npad = _EPAD - _E
  src_p = jnp.concatenate([src, jnp.full((npad,), _N, jnp.int32)])
  dst_p = jnp.concatenate([dst, jnp.zeros((npad,), jnp.int32)])
  src1_g = src_p.reshape(_G, _LANES)
  dst_g = dst_p.reshape(_G, _LANES)
  s2 = 2 * src_p
  src2_g = jnp.stack([s2, s2 + 1]).reshape(_NC, _G, _LANES)
  x16 = jnp.pad(x, ((0, 8), (0, 13)))
  W16 = jnp.pad(W_in_nbr, ((0, 13), (0, 0)))

  p0 = _seg_split(x16, src1_g, dst_g)
  h0 = _h0_call(x, p0, W_in_self, W16)
  m1 = _seg_feat(h0.reshape(2 * _NE, 16), src2_g, dst_g)
  h1 = _blk_call(h0, m1, Wb1_self, Wb1_nbr)
  m2 = _seg_feat(h1.reshape(2 * _NE, 16), src2_g, dst_g)
  h2, stats = _blk_stats_call(h1, m2, Wb2_self, Wb2_nbr)
  y, fv, off = _head_call(
      h2, stats, bn_gamma.reshape(1, _M), bn_beta.reshape(1, _M),
      W_lin, b_lin.reshape(1, 20), W_lin1, b_lin1.reshape(1, _M),
      W_c1, b_c1.reshape(1, _M), W_c2, b_c2.reshape(1, 3))
  return (y, fv, off)


# speedup vs baseline: 8.5443x; 1.0284x over previous
import functools

import jax
import jax.numpy as jnp
from jax import lax
from jax.experimental import pallas as pl
from jax.experimental.pallas import tpu as pltpu
from jax.experimental.pallas import tpu_sc as plsc

_N = 100000
_E = 1600000
_M = 32
_NC = 2
_NS = 16
_LANES = 128
_GPC = 8
_CHUNK = _LANES * _GPC
_EPAD = 1605632
_G = _EPAD // _LANES
_R = 5000
_NB = _N // _R
_NE = _N + _R
_NR = 100096
_RPT = _NR // _NS
_ZC = 368


def _make_segsum(split_edges: bool, table_rows: int):
  eps = _EPAD // (_NC * _NS) if split_edges else _EPAD // _NS
  nchunks = eps // _CHUNK
  gps = eps // _LANES
  mesh = plsc.VectorSubcoreMesh(core_axis_name="c", subcore_axis_name="s",
                                num_cores=_NC, num_subcores=_NS)

  def body(table, src_g, dst_g, out, src_v, dst_v, rows_v, acc, sem):
    c = lax.axis_index("c")
    s = lax.axis_index("s")

    def zrow(i, _):
      rows_v[i, :] = jnp.zeros((16,), jnp.float32)
      return 0
    lax.fori_loop(0, _ZC, zrow, 0)
    row0 = s * _RPT

    def zacc(j, _):
      pltpu.sync_copy(rows_v.at[pl.ds(0, _ZC)],
                      acc.at[pl.ds(row0 + j * _ZC, _ZC)])
      return 0
    lax.fori_loop(0, _RPT // _ZC, zacc, 0)
    plsc.subcore_barrier()

    gbase = (c * _NS + s) * gps if split_edges else s * gps

    def chunk(i, _):
      g0 = gbase + i * _GPC
      if split_edges:
        pltpu.sync_copy(src_g.at[pl.ds(g0, _GPC)], src_v)
      else:
        pltpu.sync_copy(src_g.at[c, pl.ds(g0, _GPC)], src_v)
      pltpu.sync_copy(dst_g.at[pl.ds(g0, _GPC)], dst_v)
      cps = [pltpu.async_copy(table.at[src_v.at[j]],
                              rows_v.at[pl.ds(j * _LANES, _LANES)], sem)
             for j in range(_GPC)]
      for j in range(_GPC):
        cps[j].wait()
      for j in range(_GPC):
        pltpu.sync_copy(rows_v.at[pl.ds(j * _LANES, _LANES)],
                        acc.at[dst_v.at[j]], add=True)
      return 0
    lax.fori_loop(0, nchunks, chunk, 0)
    plsc.subcore_barrier()
    pltpu.sync_copy(acc.at[pl.ds(row0, _RPT)], out.at[c, pl.ds(row0, _RPT)])

  idx_shape = (_G, _LANES) if split_edges else (_NC, _G, _LANES)
  del idx_shape
  return pl.kernel(
      body,
      out_type=jax.ShapeDtypeStruct((_NC, _NR, 16), jnp.float32),
      mesh=mesh,
      scratch_types=[
          pltpu.VMEM((_GPC, _LANES), jnp.int32),
          pltpu.VMEM((_GPC, _LANES), jnp.int32),
          pltpu.VMEM((_CHUNK, 16), jnp.float32),
          pltpu.VMEM_SHARED((_NR, 16), jnp.float32),
          pltpu.SemaphoreType.DMA,
      ],
      compiler_params=pltpu.CompilerParams(use_tc_tiling_on_sc=False),
  )


def _h0_body(x_ref, p_ref, ws_ref, wn_ref, h_ref):
  i = pl.program_id(0)

  @pl.when(i < _NB)
  def _():
    pb = p_ref[0] + p_ref[1]
    h_ref[...] = (
        jnp.dot(x_ref[...], ws_ref[...], preferred_element_type=jnp.float32,
                precision=lax.Precision.HIGHEST)
        + jnp.dot(pb, wn_ref[...], preferred_element_type=jnp.float32,
                precision=lax.Precision.HIGHEST))

  @pl.when(i >= _NB)
  def _():
    h_ref[...] = jnp.zeros_like(h_ref)


_h0_call = pl.pallas_call(
    _h0_body,
    grid=(_NE // _R,),
    in_specs=[
        pl.BlockSpec((_R, 3), lambda i: (jnp.minimum(i, _NB - 1), 0)),
        pl.BlockSpec((_NC, _R, 16), lambda i: (0, jnp.minimum(i, _NB - 1), 0)),
        pl.BlockSpec((3, _M), lambda i: (0, 0)),
        pl.BlockSpec((16, _M), lambda i: (0, 0)),
    ],
    out_specs=pl.BlockSpec((_R, _M), lambda i: (i, 0)),
    out_shape=jax.ShapeDtypeStruct((_NE, _M), jnp.float32),
)


def _make_blk(with_stats: bool):
  def body(h_ref, m_ref, ws_ref, wn_ref, o_ref, *rest):
    i = pl.program_id(0)
    if with_stats:
      st_ref = rest[0]

      @pl.when(i == 0)
      def _():
        st_ref[...] = jnp.zeros_like(st_ref)

    @pl.when(i < _NB)
    def _():
      hb = h_ref[...]
      m = jnp.concatenate([m_ref[0], m_ref[1]], axis=1)
      o_ref[...] = hb + jnp.maximum(
          jnp.dot(m, wn_ref[...], preferred_element_type=jnp.float32,
                precision=lax.Precision.HIGHEST)
          + jnp.dot(hb, ws_ref[...], preferred_element_type=jnp.float32,
                precision=lax.Precision.HIGHEST), 0.0)

    @pl.when(i >= _NB)
    def _():
      o_ref[...] = jnp.zeros_like(o_ref)

    if with_stats:
      ob = o_ref[...]
      ssum = jnp.sum(ob, axis=0)
      ssq = jnp.sum(ob * ob, axis=0)
      rest[0][...] += jnp.pad(jnp.stack([ssum, ssq]), ((0, 6), (0, 0)))

  in_specs = [
      pl.BlockSpec((_R, _M), lambda i: (i, 0)),
      pl.BlockSpec((_NC, _R, 16), lambda i: (0, jnp.minimum(i, _NB - 1), 0)),
      pl.BlockSpec((_M, _M), lambda i: (0, 0)),
      pl.BlockSpec((_M, _M), lambda i: (0, 0)),
  ]
  if with_stats:
    return pl.pallas_call(
        body,
        grid=(_NE // _R,),
        in_specs=in_specs,
        out_specs=[
            pl.BlockSpec((_R, _M), lambda i: (i, 0)),
            pl.BlockSpec((8, _M), lambda i: (0, 0)),
        ],
        out_shape=[
            jax.ShapeDtypeStruct((_NE, _M), jnp.float32),
            jax.ShapeDtypeStruct((8, _M), jnp.float32),
        ],
    )
  return pl.pallas_call(
      body,
      grid=(_NE // _R,),
      in_specs=in_specs,
      out_specs=pl.BlockSpec((_R, _M), lambda i: (i, 0)),
      out_shape=jax.ShapeDtypeStruct((_NE, _M), jnp.float32),
  )


def _head_body(h_ref, st_ref, g_ref, b_ref, wl_ref, bl_ref, w1_ref, b1_ref,
               wc1_ref, bc1_ref, wc2_ref, bc2_ref, y_ref, fv_ref, off_ref):
  st = st_ref[...]
  nf = jnp.float32(_N)
  mean = st[0:1, :] / nf
  var = st[1:2, :] / nf - mean * mean
  inv = lax.rsqrt(var + 1e-5)
  hb = (h_ref[...] - mean) * inv * g_ref[...] + b_ref[...]
  hb = jnp.maximum(hb, 0.0)
  y_ref[...] = jnp.dot(hb, wl_ref[...],
                       preferred_element_type=jnp.float32,
                precision=lax.Precision.HIGHEST) + bl_ref[...]
  fv = jnp.dot(hb, w1_ref[...],
               preferred_element_type=jnp.float32,
                precision=lax.Precision.HIGHEST) + b1_ref[...]
  fv_ref[...] = fv
  t = jnp.maximum(
      jnp.dot(fv, wc1_ref[...], preferred_element_type=jnp.float32,
                precision=lax.Precision.HIGHEST)
      + bc1_ref[...], 0.0)
  off_ref[...] = jnp.dot(t, wc2_ref[...],
                         preferred_element_type=jnp.float32,
                precision=lax.Precision.HIGHEST) + bc2_ref[...]


_head_call = pl.pallas_call(
    _head_body,
    grid=(_NB,),
    in_specs=[
        pl.BlockSpec((_R, _M), lambda i: (i, 0)),
        pl.BlockSpec((8, _M), lambda i: (0, 0)),
        pl.BlockSpec((1, _M), lambda i: (0, 0)),
        pl.BlockSpec((1, _M), lambda i: (0, 0)),
        pl.BlockSpec((_M, 20), lambda i: (0, 0)),
        pl.BlockSpec((1, 20), lambda i: (0, 0)),
        pl.BlockSpec((_M, _M), lambda i: (0, 0)),
        pl.BlockSpec((1, _M), lambda i: (0, 0)),
        pl.BlockSpec((_M, _M), lambda i: (0, 0)),
        pl.BlockSpec((1, _M), lambda i: (0, 0)),
        pl.BlockSpec((_M, 3), lambda i: (0, 0)),
        pl.BlockSpec((1, 3), lambda i: (0, 0)),
    ],
    out_specs=[
        pl.BlockSpec((_R, 20), lambda i: (i, 0)),
        pl.BlockSpec((_R, _M), lambda i: (i, 0)),
        pl.BlockSpec((_R, 3), lambda i: (i, 0)),
    ],
    out_shape=[
        jax.ShapeDtypeStruct((_N, 20), jnp.float32),
        jax.ShapeDtypeStruct((_N, _M), jnp.float32),
        jax.ShapeDtypeStruct((_N, 3), jnp.float32),
    ],
)

_make_segsum = functools.lru_cache(maxsize=None)(_make_segsum)


def _seg_split(table, src_g, dst_g):
  return _make_segsum(True, _N + 8)(table, src_g, dst_g)


def _seg_feat(table, src_g, dst_g):
  return _make_segsum(False, 2 * _NE)(table, src_g, dst_g)


_blk_call = _make_blk(with_stats=False)
_blk_stats_call = _make_blk(with_stats=True)


def kernel(x, edge_index, W_in_self, W_in_nbr, Wb1_self, Wb1_nbr, Wb2_self,
           Wb2_nbr, bn_gamma, bn_beta, W_lin, b_lin, W_lin1, b_lin1, W_c1,
           b_c1, W_c2, b_c2):
  src = edge_index[0]
  dst = edge_index[1]
  npad = _EPAD - _E
  src_p = jnp.concatenate([src, jnp.full((npad,), _N, jnp.int32)])
  dst_p = jnp.concatenate([dst, jnp.zeros((npad,), jnp.int32)])
  src1_g = src_p.reshape(_G, _LANES)
  dst_g = dst_p.reshape(_G, _LANES)
  s2 = 2 * src_p
  src2_g = jnp.stack([s2, s2 + 1]).reshape(_NC, _G, _LANES)
  x16 = jnp.pad(x, ((0, 8), (0, 13)))
  W16 = jnp.pad(W_in_nbr, ((0, 13), (0, 0)))

  p0 = _seg_split(x16, src1_g, dst_g)
  h0 = _h0_call(x, p0, W_in_self, W16)
  m1 = _seg_feat(h0.reshape(2 * _NE, 16), src2_g, dst_g)
  h1 = _blk_call(h0, m1, Wb1_self, Wb1_nbr)
  m2 = _seg_feat(h1.reshape(2 * _NE, 16), src2_g, dst_g)
  h2, stats = _blk_stats_call(h1, m2, Wb2_self, Wb2_nbr)
  y, fv, off = _head_call(
      h2, stats, bn_gamma.reshape(1, _M), bn_beta.reshape(1, _M),
      W_lin, b_lin.reshape(1, 20), W_lin1, b_lin1.reshape(1, _M),
      W_c1, b_c1.reshape(1, _M), W_c2, b_c2.reshape(1, 3))
  return (y, fv, off)


# speedup vs baseline: 13.1529x; 1.5394x over previous
import functools

import jax
import jax.numpy as jnp
from jax import lax
from jax.experimental import pallas as pl
from jax.experimental.pallas import tpu as pltpu
from jax.experimental.pallas import tpu_sc as plsc

_N = 100000
_E = 1600000
_M = 32
_NC = 2
_NS = 16
_LANES = 128
_GPC = 8
_CHUNK = _LANES * _GPC
_EPAD = 1605632
_G = _EPAD // _LANES
_NP = 102400
_Q = _NP // 8
_QB = 512
_GB = _Q // _QB
_NR = _NP
_RPT = _NR // _NS
_ZC = 400


def _make_segsum(split_edges: bool, table_rows: int):
  eps = _EPAD // (_NC * _NS) if split_edges else _EPAD // _NS
  nchunks = eps // _CHUNK
  gps = eps // _LANES
  mesh = plsc.VectorSubcoreMesh(core_axis_name="c", subcore_axis_name="s",
                                num_cores=_NC, num_subcores=_NS)

  def body(table, src_g, dst_g, out, src_v, dst_v, rows_v, acc, sem):
    c = lax.axis_index("c")
    s = lax.axis_index("s")

    def zrow(i, _):
      rows_v[i, :] = jnp.zeros((16,), jnp.float32)
      return 0
    lax.fori_loop(0, _ZC, zrow, 0)
    row0 = s * _RPT

    def zacc(j, _):
      pltpu.sync_copy(rows_v.at[pl.ds(0, _ZC)],
                      acc.at[pl.ds(row0 + j * _ZC, _ZC)])
      return 0
    lax.fori_loop(0, _RPT // _ZC, zacc, 0)
    plsc.subcore_barrier()

    gbase = (c * _NS + s) * gps if split_edges else s * gps

    def chunk(i, _):
      g0 = gbase + i * _GPC
      if split_edges:
        pltpu.sync_copy(src_g.at[pl.ds(g0, _GPC)], src_v)
      else:
        pltpu.sync_copy(src_g.at[c, pl.ds(g0, _GPC)], src_v)
      pltpu.sync_copy(dst_g.at[pl.ds(g0, _GPC)], dst_v)
      cps = [pltpu.async_copy(table.at[src_v.at[j]],
                              rows_v.at[pl.ds(j * _LANES, _LANES)], sem)
             for j in range(_GPC)]
      for j in range(_GPC):
        cps[j].wait()
      for j in range(_GPC):
        pltpu.sync_copy(rows_v.at[pl.ds(j * _LANES, _LANES)],
                        acc.at[dst_v.at[j]], add=True)
      return 0
    lax.fori_loop(0, nchunks, chunk, 0)
    plsc.subcore_barrier()
    pltpu.sync_copy(acc.at[pl.ds(row0, _RPT)], out.at[c, pl.ds(row0, _RPT)])

  del table_rows
  return pl.kernel(
      body,
      out_type=jax.ShapeDtypeStruct((_NC, _NR, 16), jnp.float32),
      mesh=mesh,
      scratch_types=[
          pltpu.VMEM((_GPC, _LANES), jnp.int32),
          pltpu.VMEM((_GPC, _LANES), jnp.int32),
          pltpu.VMEM((_CHUNK, 16), jnp.float32),
          pltpu.VMEM_SHARED((_NR, 16), jnp.float32),
          pltpu.SemaphoreType.DMA,
      ],
      compiler_params=pltpu.CompilerParams(use_tc_tiling_on_sc=False),
  )


def _dot(a, b):
  return jnp.dot(a, b, preferred_element_type=jnp.float32,
                 precision=lax.Precision.HIGHEST)


def _h0_body(x_ref, p_ref, w_ref, hl_ref, hr_ref):
  pb = p_ref[0] + p_ref[1]
  xb = x_ref[...]
  hl_ref[...] = _dot(xb, w_ref[0]) + _dot(pb, w_ref[2])
  hr_ref[...] = _dot(xb, w_ref[1]) + _dot(pb, w_ref[3])


_h0_call = pl.pallas_call(
    _h0_body,
    grid=(_GB,),
    in_specs=[
        pl.BlockSpec((_QB, 128), lambda i: (i, 0)),
        pl.BlockSpec((_NC, _QB, 128), lambda i: (0, i, 0)),
        pl.BlockSpec((4, 128, 128), lambda i: (0, 0, 0)),
    ],
    out_specs=[
        pl.BlockSpec((_QB, 128), lambda i: (i, 0)),
        pl.BlockSpec((_QB, 128), lambda i: (i, 0)),
    ],
    out_shape=[
        jax.ShapeDtypeStruct((_Q, 128), jnp.float32),
        jax.ShapeDtypeStruct((_Q, 128), jnp.float32),
    ],
)


def _make_blk(with_stats: bool):
  def body(hl_ref, hr_ref, m_ref, w_ref, ol_ref, or_ref, *rest):
    hl = hl_ref[...]
    hr = hr_ref[...]
    ml = m_ref[0]
    mr = m_ref[1]
    tl = (_dot(ml, w_ref[0]) + _dot(mr, w_ref[1])
          + _dot(hl, w_ref[2]) + _dot(hr, w_ref[3]))
    tr = (_dot(ml, w_ref[4]) + _dot(mr, w_ref[5])
          + _dot(hl, w_ref[6]) + _dot(hr, w_ref[7]))
    ol = hl + jnp.maximum(tl, 0.0)
    orr = hr + jnp.maximum(tr, 0.0)
    ol_ref[...] = ol
    or_ref[...] = orr
    if with_stats:
      st_ref = rest[0]

      @pl.when(pl.program_id(0) == 0)
      def _():
        st_ref[...] = jnp.zeros_like(st_ref)

      sums = jnp.stack([jnp.sum(ol, axis=0), jnp.sum(orr, axis=0),
                        jnp.sum(ol * ol, axis=0), jnp.sum(orr * orr, axis=0)])
      st_ref[...] += jnp.pad(sums, ((0, 4), (0, 0)))

  in_specs = [
      pl.BlockSpec((_QB, 128), lambda i: (i, 0)),
      pl.BlockSpec((_QB, 128), lambda i: (i, 0)),
      pl.BlockSpec((_NC, _QB, 128), lambda i: (0, i, 0)),
      pl.BlockSpec((8, 128, 128), lambda i: (0, 0, 0)),
  ]
  half = jax.ShapeDtypeStruct((_Q, 128), jnp.float32)
  if with_stats:
    return pl.pallas_call(
        body,
        grid=(_GB,),
        in_specs=in_specs,
        out_specs=[
            pl.BlockSpec((_QB, 128), lambda i: (i, 0)),
            pl.BlockSpec((_QB, 128), lambda i: (i, 0)),
            pl.BlockSpec((8, 128), lambda i: (0, 0)),
        ],
        out_shape=[half, half, jax.ShapeDtypeStruct((8, 128), jnp.float32)],
    )
  return pl.pallas_call(
      body,
      grid=(_GB,),
      in_specs=in_specs,
      out_specs=[
          pl.BlockSpec((_QB, 128), lambda i: (i, 0)),
          pl.BlockSpec((_QB, 128), lambda i: (i, 0)),
      ],
      out_shape=[half, half],
  )


def _head_body(hl_ref, hr_ref, ss_ref, wll_ref, wlr_ref, by_ref, w1l_ref,
               w1r_ref, b1_ref, wc1_ref, bc1_ref, wc2_ref, bc2_ref,
               y_ref, fv_ref, off_ref):
  hbl = jnp.maximum(hl_ref[...] * ss_ref[0:1] + ss_ref[1:2], 0.0)
  hbr = jnp.maximum(hr_ref[...] * ss_ref[2:3] + ss_ref[3:4], 0.0)
  y_ref[...] = _dot(hbl, wll_ref[...]) + _dot(hbr, wlr_ref[...]) + by_ref[...]
  fv = _dot(hbl, w1l_ref[...]) + _dot(hbr, w1r_ref[...]) + b1_ref[...]
  fv_ref[...] = fv
  t = jnp.maximum(_dot(fv, wc1_ref[...]) + bc1_ref[...], 0.0)
  off_ref[...] = _dot(t, wc2_ref[...]) + bc2_ref[...]


_head_call = pl.pallas_call(
    _head_body,
    grid=(_GB,),
    in_specs=[
        pl.BlockSpec((_QB, 128), lambda i: (i, 0)),
        pl.BlockSpec((_QB, 128), lambda i: (i, 0)),
        pl.BlockSpec((4, 128), lambda i: (0, 0)),
        pl.BlockSpec((128, 160), lambda i: (0, 0)),
        pl.BlockSpec((128, 160), lambda i: (0, 0)),
        pl.BlockSpec((1, 160), lambda i: (0, 0)),
        pl.BlockSpec((128, 256), lambda i: (0, 0)),
        pl.BlockSpec((128, 256), lambda i: (0, 0)),
        pl.BlockSpec((1, 256), lambda i: (0, 0)),
        pl.BlockSpec((256, 256), lambda i: (0, 0)),
        pl.BlockSpec((1, 256), lambda i: (0, 0)),
        pl.BlockSpec((256, 24), lambda i: (0, 0)),
        pl.BlockSpec((1, 24), lambda i: (0, 0)),
    ],
    out_specs=[
        pl.BlockSpec((_QB, 160), lambda i: (i, 0)),
        pl.BlockSpec((_QB, 256), lambda i: (i, 0)),
        pl.BlockSpec((_QB, 24), lambda i: (i, 0)),
    ],
    out_shape=[
        jax.ShapeDtypeStruct((_Q, 160), jnp.float32),
        jax.ShapeDtypeStruct((_Q, 256), jnp.float32),
        jax.ShapeDtypeStruct((_Q, 24), jnp.float32),
    ],
)

_make_segsum = functools.lru_cache(maxsize=None)(_make_segsum)


def _seg_split(table, src_g, dst_g):
  return _make_segsum(True, _NP)(table, src_g, dst_g)


def _seg_feat(table, src_g, dst_g):
  return _make_segsum(False, 2 * _NP)(table, src_g, dst_g)


_blk_call = _make_blk(with_stats=False)
_blk_stats_call = _make_blk(with_stats=True)


def _bd8(a):
  return jnp.kron(jnp.eye(8, dtype=a.dtype), a)


def kernel(x, edge_index, W_in_self, W_in_nbr, Wb1_self, Wb1_nbr, Wb2_self,
           Wb2_nbr, bn_gamma, bn_beta, W_lin, b_lin, W_lin1, b_lin1, W_c1,
           b_c1, W_c2, b_c2):
  src = edge_index[0]
  dst = edge_index[1]
  npad = _EPAD - _E
  src_p = jnp.concatenate([src, jnp.full((npad,), _N, jnp.int32)])
  dst_p = jnp.concatenate([dst, jnp.zeros((npad,), jnp.int32)])
  src1_g = src_p.reshape(_G, _LANES)
  dst_g = dst_p.reshape(_G, _LANES)
  src2_g = jnp.stack([src_p, src_p + _NP]).reshape(_NC, _G, _LANES)

  x16p = jnp.pad(x, ((0, _NP - _N), (0, 13))).reshape(_Q, 128)

  W16s = jnp.pad(W_in_self, ((0, 13), (0, 0)))
  W16n = jnp.pad(W_in_nbr, ((0, 13), (0, 0)))
  wh0 = jnp.stack([_bd8(W16s[:, :16]), _bd8(W16s[:, 16:]),
                   _bd8(W16n[:, :16]), _bd8(W16n[:, 16:])])

  def blkw(Ws, Wn):
    return jnp.stack([
        _bd8(Wn[:16, :16]), _bd8(Wn[16:, :16]),
        _bd8(Ws[:16, :16]), _bd8(Ws[16:, :16]),
        _bd8(Wn[:16, 16:]), _bd8(Wn[16:, 16:]),
        _bd8(Ws[:16, 16:]), _bd8(Ws[16:, 16:])])

  w1 = blkw(Wb1_self, Wb1_nbr)
  w2 = blkw(Wb2_self, Wb2_nbr)

  p0 = _seg_split(x16p.reshape(_NP, 16), src1_g, dst_g)
  h0l, h0r = _h0_call(x16p, p0.reshape(_NC, _Q, 128), wh0)
  tab1 = jnp.concatenate([h0l, h0r]).reshape(2 * _NP, 16)
  m1 = _seg_feat(tab1, src2_g, dst_g)
  h1l, h1r = _blk_call(h0l, h0r, m1.reshape(_NC, _Q, 128), w1)
  tab2 = jnp.concatenate([h1l, h1r]).reshape(2 * _NP, 16)
  m2 = _seg_feat(tab2, src2_g, dst_g)
  h2l, h2r, st = _blk_stats_call(h1l, h1r, m2.reshape(_NC, _Q, 128), w2)

  nf = jnp.float32(_N)
  mean = jnp.concatenate([st[0].reshape(8, 16).sum(0),
                          st[1].reshape(8, 16).sum(0)]) / nf
  sq = jnp.concatenate([st[2].reshape(8, 16).sum(0),
                        st[3].reshape(8, 16).sum(0)]) / nf
  inv = lax.rsqrt(sq - mean * mean + 1e-5)
  scale = inv * bn_gamma
  shift = bn_beta - mean * scale
  ss = jnp.stack([jnp.tile(scale[:16], 8), jnp.tile(shift[:16], 8),
                  jnp.tile(scale[16:], 8), jnp.tile(shift[16:], 8)])

  yp, fvp, offp = _head_call(
      h2l, h2r, ss,
      _bd8(W_lin[:16, :]), _bd8(W_lin[16:, :]), jnp.tile(b_lin, 8)[None, :],
      _bd8(W_lin1[:16, :]), _bd8(W_lin1[16:, :]),
      jnp.tile(b_lin1, 8)[None, :],
      _bd8(W_c1), jnp.tile(b_c1, 8)[None, :],
      _bd8(W_c2), jnp.tile(b_c2, 8)[None, :])
  y = yp.reshape(_NP, 20)[:_N]
  fv = fvp.reshape(_NP, _M)[:_N]
  off = offp.reshape(_NP, 3)[:_N]
  return (y, fv, off)
